# baseline (speedup 1.0000x reference)
import jax
import jax.numpy as jnp
from jax import lax
from jax.experimental import pallas as pl
from jax.experimental.pallas import tpu as pltpu
from jax.experimental.pallas import tpu_sc as plsc

N_TEAMS = 100000
BATCH = 16384
NC = 2
NS = 16
L = 16
NW = NC * NS
B_PER_W = BATCH // NW


def _body(home_hbm, away_hbm, att_hbm, dfn_hbm, adv_hbm,
          lamh_hbm, lama_hbm,
          hidx_v, aidx_v, ha_v, ad_v, aa_v, hd_v, lamh_v, lama_v, adv_v,
          semh, sema, semv, semg1, semg2, semo):
    wid = lax.axis_index("s") * NC + lax.axis_index("c")
    base = wid * B_PER_W

    ch = pltpu.async_copy(home_hbm.at[pl.ds(base, B_PER_W)], hidx_v, semh)
    ca = pltpu.async_copy(away_hbm.at[pl.ds(base, B_PER_W)], aidx_v, sema)
    cv = pltpu.async_copy(adv_hbm, adv_v.at[pl.ds(0, 1)], semv)
    ch.wait()
    c1 = pltpu.async_copy(att_hbm.at[hidx_v], ha_v, semg1)
    ca.wait()
    c2 = pltpu.async_copy(dfn_hbm.at[aidx_v], ad_v, semg1)
    c3 = pltpu.async_copy(att_hbm.at[aidx_v], aa_v, semg2)
    c4 = pltpu.async_copy(dfn_hbm.at[hidx_v], hd_v, semg2)
    cv.wait()
    adv = jnp.broadcast_to(adv_v[...][0], (L,))

    c1.wait()
    c2.wait()

    def _loop1(i, carry):
        s = pl.ds(i * L, L)
        lamh_v[s] = jnp.exp(ha_v[s] - ad_v[s] + adv)
        return carry

    lax.fori_loop(0, B_PER_W // L, _loop1, 0, unroll=2)
    o1 = pltpu.async_copy(lamh_v, lamh_hbm.at[pl.ds(base, B_PER_W)], semo)

    c3.wait()
    c4.wait()

    def _loop2(i, carry):
        s = pl.ds(i * L, L)
        lama_v[s] = jnp.exp(aa_v[s] - hd_v[s])
        return carry

    lax.fori_loop(0, B_PER_W // L, _loop2, 0, unroll=2)
    o2 = pltpu.async_copy(lama_v, lama_hbm.at[pl.ds(base, B_PER_W)], semo)
    o1.wait()
    o2.wait()


def _sc_call(home_i32, away_i32, attack, defense, adv_vec):
    mesh = plsc.VectorSubcoreMesh(
        core_axis_name="c", subcore_axis_name="s",
        num_cores=NC, num_subcores=NS)
    f32 = jnp.float32
    out_type = (jax.ShapeDtypeStruct((BATCH,), f32),
                jax.ShapeDtypeStruct((BATCH,), f32))
    scratch = [
        pltpu.VMEM((B_PER_W,), jnp.int32),
        pltpu.VMEM((B_PER_W,), jnp.int32),
        pltpu.VMEM((B_PER_W,), f32),
        pltpu.VMEM((B_PER_W,), f32),
        pltpu.VMEM((B_PER_W,), f32),
        pltpu.VMEM((B_PER_W,), f32),
        pltpu.VMEM((B_PER_W,), f32),
        pltpu.VMEM((B_PER_W,), f32),
        pltpu.VMEM((L,), f32),
        pltpu.SemaphoreType.DMA,
        pltpu.SemaphoreType.DMA,
        pltpu.SemaphoreType.DMA,
        pltpu.SemaphoreType.DMA,
        pltpu.SemaphoreType.DMA,
        pltpu.SemaphoreType.DMA,
    ]
    return pl.kernel(_body, out_type=out_type, mesh=mesh,
                     scratch_types=scratch)(
        home_i32, away_i32, attack, defense, adv_vec)


def kernel(home_teams, away_teams, attack_ratings, defense_ratings,
           home_advantage):
    home_i32 = home_teams.astype(jnp.int32)
    away_i32 = away_teams.astype(jnp.int32)
    adv_vec = jnp.asarray(home_advantage, dtype=jnp.float32).reshape(1)
    return _sc_call(home_i32, away_i32, attack_ratings, defense_ratings,
                    adv_vec)


# speedup vs baseline: 1.0349x; 1.0349x over previous
import jax
import jax.numpy as jnp
from jax import lax
from jax.experimental import pallas as pl
from jax.experimental.pallas import tpu as pltpu
from jax.experimental.pallas import tpu_sc as plsc

N_TEAMS = 100000
BATCH = 16384
NC = 2
NS = 16
L = 16
NW = NC * NS
B_PER_W = BATCH // NW


def _body(home_hbm, away_hbm, att_hbm, dfn_hbm, adv_hbm,
          lamh_hbm, lama_hbm,
          hidx_v, aidx_v, ha_v, ad_v, aa_v, hd_v, lamh_v, lama_v, adv_v,
          semh, sema, semv, semg1, semg2, semo):
    wid = lax.axis_index("s") * NC + lax.axis_index("c")
    base = wid * B_PER_W

    ch = pltpu.async_copy(home_hbm.at[pl.ds(base, B_PER_W)], hidx_v, semh)
    ca = pltpu.async_copy(away_hbm.at[pl.ds(base, B_PER_W)], aidx_v, sema)
    cv = pltpu.async_copy(adv_hbm, adv_v.at[pl.ds(0, 1)], semv)
    ch.wait()
    c1 = pltpu.async_copy(att_hbm.at[hidx_v], ha_v, semg1)
    ca.wait()
    c2 = pltpu.async_copy(dfn_hbm.at[aidx_v], ad_v, semg1)
    c3 = pltpu.async_copy(att_hbm.at[aidx_v], aa_v, semg2)
    c4 = pltpu.async_copy(dfn_hbm.at[hidx_v], hd_v, semg2)
    cv.wait()
    adv = jnp.broadcast_to(adv_v[...][0], (L,))

    c1.wait()
    c2.wait()

    def _loop1(i, carry):
        s = pl.ds(i * L, L)
        lamh_v[s] = jnp.exp(ha_v[s] - ad_v[s] + adv)
        return carry

    lax.fori_loop(0, B_PER_W // L, _loop1, 0, unroll=1)
    o1 = pltpu.async_copy(lamh_v, lamh_hbm.at[pl.ds(base, B_PER_W)], semo)

    c3.wait()
    c4.wait()

    def _loop2(i, carry):
        s = pl.ds(i * L, L)
        lama_v[s] = jnp.exp(aa_v[s] - hd_v[s])
        return carry

    lax.fori_loop(0, B_PER_W // L, _loop2, 0, unroll=1)
    o2 = pltpu.async_copy(lama_v, lama_hbm.at[pl.ds(base, B_PER_W)], semo)
    o1.wait()
    o2.wait()


def _sc_call(home_i32, away_i32, attack, defense, adv_vec):
    mesh = plsc.VectorSubcoreMesh(
        core_axis_name="c", subcore_axis_name="s",
        num_cores=NC, num_subcores=NS)
    f32 = jnp.float32
    out_type = (jax.ShapeDtypeStruct((BATCH,), f32),
                jax.ShapeDtypeStruct((BATCH,), f32))
    scratch = [
        pltpu.VMEM((B_PER_W,), jnp.int32),
        pltpu.VMEM((B_PER_W,), jnp.int32),
        pltpu.VMEM((B_PER_W,), f32),
        pltpu.VMEM((B_PER_W,), f32),
        pltpu.VMEM((B_PER_W,), f32),
        pltpu.VMEM((B_PER_W,), f32),
        pltpu.VMEM((B_PER_W,), f32),
        pltpu.VMEM((B_PER_W,), f32),
        pltpu.VMEM((L,), f32),
        pltpu.SemaphoreType.DMA,
        pltpu.SemaphoreType.DMA,
        pltpu.SemaphoreType.DMA,
        pltpu.SemaphoreType.DMA,
        pltpu.SemaphoreType.DMA,
        pltpu.SemaphoreType.DMA,
    ]
    return pl.kernel(_body, out_type=out_type, mesh=mesh,
                     scratch_types=scratch)(
        home_i32, away_i32, attack, defense, adv_vec)


def kernel(home_teams, away_teams, attack_ratings, defense_ratings,
           home_advantage):
    home_i32 = home_teams.astype(jnp.int32)
    away_i32 = away_teams.astype(jnp.int32)
    adv_vec = jnp.asarray(home_advantage, dtype=jnp.float32).reshape(1)
    return _sc_call(home_i32, away_i32, attack_ratings, defense_ratings,
                    adv_vec)
